# Initial kernel scaffold; baseline (speedup 1.0000x reference)
import jax
import jax.numpy as jnp
from jax.experimental import pallas as pl


def kernel(x, edge_index, edge_factor, W0, b0, W1, b1):
    raise NotImplementedError("write your pallas kernel here")


# speedup vs baseline: 3.3132x; 3.3132x over previous
import functools

import jax
import jax.numpy as jnp
from jax import lax
from jax.experimental import pallas as pl
from jax.experimental.pallas import tpu as pltpu
from jax.experimental.pallas import tpu_sc as plsc

N = 10000
E = 320000
D = 128
OUT_D = 128
BETA = 0.1

NC = 2
NS = 16
CHUNK = 128
NCH = -(-E // (NC * NS * CHUNK))
EP = NC * NS * NCH * CHUNK
NP = 10240
RPT = NP // NS

_MESH = plsc.VectorSubcoreMesh(
    core_axis_name="c", subcore_axis_name="s", num_cores=NC, num_subcores=NS)


def _fill_zero_rows(ref):
  z16 = jnp.zeros((16,), jnp.float32)

  def body(i, carry):
    for l in range(8):
      ref[i, pl.ds(l * 16, 16)] = z16
    return carry

  lax.fori_loop(0, ref.shape[0], body, 0)


@functools.partial(
    pl.kernel,
    out_type=jax.ShapeDtypeStruct((NC, NP), jnp.float32),
    mesh=_MESH,
    scratch_types=[
        pltpu.VMEM((NCH, CHUNK), jnp.int32),
        pltpu.VMEM((NCH, CHUNK), jnp.float32),
        pltpu.VMEM((RPT,), jnp.float32),
        pltpu.VMEM_SHARED((NP,), jnp.float32),
    ],
)
def _deg_kernel(dst_hbm, ones_hbm, out_hbm, dst_v, ones_v, zbuf, acc):
  c = lax.axis_index("c")
  s = lax.axis_index("s")

  pltpu.sync_copy(dst_hbm.at[c, s], dst_v)
  pltpu.sync_copy(ones_hbm.at[c, s], ones_v)

  z16 = jnp.zeros((16,), jnp.float32)

  def zbody(i, carry):
    zbuf[pl.ds(i * 16, 16)] = z16
    return carry

  lax.fori_loop(0, RPT // 16, zbody, 0)
  pltpu.sync_copy(zbuf, acc.at[pl.ds(s * RPT, RPT)])
  plsc.subcore_barrier()

  def body(j, carry):
    pltpu.sync_copy(ones_v.at[j], acc.at[dst_v.at[j]], add=True)
    return carry

  lax.fori_loop(0, NCH, body, 0)
  plsc.subcore_barrier()
  pltpu.sync_copy(acc.at[pl.ds(s * RPT, RPT)], out_hbm.at[c, pl.ds(s * RPT, RPT)])


@functools.partial(
    pl.kernel,
    out_type=jax.ShapeDtypeStruct((NC, NP, D), jnp.float32),
    mesh=_MESH,
    scratch_types=[
        pltpu.VMEM((NCH, CHUNK), jnp.int32),
        pltpu.VMEM((NCH, CHUNK), jnp.int32),
        pltpu.VMEM((NCH, CHUNK), jnp.float32),
        pltpu.VMEM((CHUNK, D), jnp.float32),
        pltpu.VMEM_SHARED((NP, D), jnp.float32),
    ],
)
def _prop_kernel(z_hbm, src_hbm, dst_hbm, w_hbm, out_hbm,
                 src_v, dst_v, w_v, rows_v, acc):
  c = lax.axis_index("c")
  s = lax.axis_index("s")

  pltpu.sync_copy(src_hbm.at[c, s], src_v)
  pltpu.sync_copy(dst_hbm.at[c, s], dst_v)
  pltpu.sync_copy(w_hbm.at[c, s], w_v)

  _fill_zero_rows(rows_v)
  for k in range(RPT // CHUNK):
    pltpu.sync_copy(rows_v, acc.at[pl.ds(s * RPT + k * CHUNK, CHUNK)])
  plsc.subcore_barrier()

  def chunk_body(j, carry):
    pltpu.sync_copy(z_hbm.at[src_v.at[j]], rows_v)

    def row_body(i16, rcarry):
      fv = w_v[j, pl.ds(i16 * 16, 16)]
      for r in range(16):
        f = fv[r]
        row = i16 * 16 + r
        for l in range(8):
          sl = (row, pl.ds(l * 16, 16))
          rows_v[sl] = rows_v[sl] * f
      return rcarry

    lax.fori_loop(0, CHUNK // 16, row_body, 0)

    pltpu.sync_copy(rows_v, acc.at[dst_v.at[j]], add=True)
    return carry

  lax.fori_loop(0, NCH, chunk_body, 0)
  plsc.subcore_barrier()

  for k in range(RPT // CHUNK):
    r0 = s * RPT + k * CHUNK
    pltpu.sync_copy(acc.at[pl.ds(r0, CHUNK)], out_hbm.at[c, pl.ds(r0, CHUNK)])


_RB = 256


def _prep_body(d0_ref, d1_ref, x_ref, z_ref, n_ref):
  deg = d0_ref[...] + d1_ref[...]
  norm = lax.rsqrt(jnp.maximum(deg, 1.0))
  n_ref[...] = norm
  z_ref[...] = x_ref[...] * norm


def _prep(d0, d1, x):
  return pl.pallas_call(
      _prep_body,
      grid=(NP // _RB,),
      in_specs=[
          pl.BlockSpec((_RB, 1), lambda i: (i, 0)),
          pl.BlockSpec((_RB, 1), lambda i: (i, 0)),
          pl.BlockSpec((_RB, D), lambda i: (i, 0)),
      ],
      out_specs=[
          pl.BlockSpec((_RB, D), lambda i: (i, 0)),
          pl.BlockSpec((_RB, 1), lambda i: (i, 0)),
      ],
      out_shape=[
          jax.ShapeDtypeStruct((NP, D), jnp.float32),
          jax.ShapeDtypeStruct((NP, 1), jnp.float32),
      ],
  )(d0, d1, x)


def _combine_body(p_ref, x_ref, n_ref, z_ref):
  agg = p_ref[0] + p_ref[1]
  h = agg * (1.0 - BETA) + x_ref[...] * BETA
  z_ref[...] = h * n_ref[...]


def _combine(p, x, ncol):
  return pl.pallas_call(
      _combine_body,
      grid=(NP // _RB,),
      in_specs=[
          pl.BlockSpec((NC, _RB, D), lambda i: (0, i, 0)),
          pl.BlockSpec((_RB, D), lambda i: (i, 0)),
          pl.BlockSpec((_RB, 1), lambda i: (i, 0)),
      ],
      out_specs=pl.BlockSpec((_RB, D), lambda i: (i, 0)),
      out_shape=jax.ShapeDtypeStruct((NP, D), jnp.float32),
  )(p, x, ncol)


def _final_body(q0_ref, q1_ref, x_ref, w0_ref, w1_ref, b0_ref, b1_ref, o_ref):
  x = x_ref[...]
  dn = (((1,), (1,)), ((), ()))
  h0 = (q0_ref[0] + q0_ref[1]) * (1.0 - BETA) + x * BETA
  o0 = lax.dot_general(h0, w0_ref[...], dn, preferred_element_type=jnp.float32)
  o0 = jnp.maximum(o0 + b0_ref[...], 0.0)
  h1 = (q1_ref[0] + q1_ref[1]) * (1.0 - BETA) + x * BETA
  o1 = lax.dot_general(h1, w1_ref[...], dn, preferred_element_type=jnp.float32)
  o1 = jnp.maximum(o1 + b1_ref[...], 0.0)
  o_ref[...] = jnp.concatenate([o0, o1], axis=1)


def _final(q0, q1, x, W0, W1, b0, b1):
  return pl.pallas_call(
      _final_body,
      grid=(NP // _RB,),
      in_specs=[
          pl.BlockSpec((NC, _RB, D), lambda i: (0, i, 0)),
          pl.BlockSpec((NC, _RB, D), lambda i: (0, i, 0)),
          pl.BlockSpec((_RB, D), lambda i: (i, 0)),
          pl.BlockSpec((OUT_D, D), lambda i: (0, 0)),
          pl.BlockSpec((OUT_D, D), lambda i: (0, 0)),
          pl.BlockSpec((1, OUT_D), lambda i: (0, 0)),
          pl.BlockSpec((1, OUT_D), lambda i: (0, 0)),
      ],
      out_specs=pl.BlockSpec((_RB, 2 * OUT_D), lambda i: (i, 0)),
      out_shape=jax.ShapeDtypeStruct((NP, 2 * OUT_D), jnp.float32),
  )(q0, q1, x, W0, W1, b0, b1)


def kernel(x, edge_index, edge_factor, W0, b0, W1, b1):
  pad = EP - E
  shape4 = (NC, NS, NCH, CHUNK)
  src_r = jnp.concatenate(
      [edge_index[0], jnp.zeros((pad,), jnp.int32)]).reshape(shape4)
  dst_r = jnp.concatenate(
      [edge_index[1], jnp.zeros((pad,), jnp.int32)]).reshape(shape4)
  zpad = jnp.zeros((pad,), jnp.float32)
  w0_r = jnp.concatenate([edge_factor[0], zpad]).reshape(shape4)
  w1_r = jnp.concatenate([edge_factor[1], zpad]).reshape(shape4)
  ones_r = jnp.concatenate([jnp.ones((E,), jnp.float32), zpad]).reshape(shape4)
  x_p = jnp.pad(x, ((0, NP - N), (0, 0)))

  deg2 = _deg_kernel(dst_r, ones_r)
  z0, ncol = _prep(deg2[0][:, None], deg2[1][:, None], x_p)

  p0 = _prop_kernel(z0, src_r, dst_r, w0_r)
  z1_0 = _combine(p0, x_p, ncol)
  q0 = _prop_kernel(z1_0, src_r, dst_r, w0_r)

  p1 = _prop_kernel(z0, src_r, dst_r, w1_r)
  z1_1 = _combine(p1, x_p, ncol)
  q1 = _prop_kernel(z1_1, src_r, dst_r, w1_r)

  out = _final(q0, q1, x_p, W0, W1, b0[None, :], b1[None, :])
  return out[:N]
